# baseline (speedup 1.0000x reference)
import functools

import jax
import jax.numpy as jnp
from jax import lax
from jax.experimental import pallas as pl
from jax.experimental.pallas import tpu as pltpu
from jax.experimental.pallas import tpu_sc as plsc

NUM_FIELDS = 26
VOCAB = 100000
EMB = 32
B = 16384
NUM_NUM = 13
EPS = 1e-5

NC = 2
NS = 16
NW = NC * NS
ROWS = NUM_FIELDS * EMB
RPW = ROWS // NW
NQ = 4
QTR = B // NQ
LANES = 16


@functools.cache
def _make_sc_lookup():
    mesh = plsc.VectorSubcoreMesh(
        core_axis_name="c", subcore_axis_name="s", num_cores=NC, num_subcores=NS
    )

    @functools.partial(
        pl.kernel,
        out_type=jax.ShapeDtypeStruct((ROWS, B), jnp.float32),
        mesh=mesh,
        scratch_types=[
            pltpu.VMEM((VOCAB,), jnp.float32),
            pltpu.VMEM((B,), jnp.int32),
            pltpu.VMEM((QTR,), jnp.float32),
            pltpu.VMEM((QTR,), jnp.float32),
            pltpu.SemaphoreType.DMA,
            pltpu.SemaphoreType.DMA,
            pltpu.SemaphoreType.DMA,
        ],
        compiler_params=pltpu.CompilerParams(
            use_tc_tiling_on_sc=True, needs_layout_passes=False
        ),
    )
    def _sc_lookup(tabT_hbm, xcatT_hbm, out_hbm, row_v, idx_v,
                   out_v0, out_v1, sem0, sem1, sem2):
        wid = lax.axis_index("s") * NC + lax.axis_index("c")
        r0 = wid * RPW
        outs = (out_v0, out_v1)
        sems = (sem0, sem1)
        desc = [None, None]
        for j in range(RPW):
            r = r0 + j
            f = r // EMB
            if j == 0:
                pltpu.sync_copy(xcatT_hbm.at[f], idx_v)
            else:
                fprev = (r - 1) // EMB

                @pl.when(f != fprev)
                def _():
                    pltpu.sync_copy(xcatT_hbm.at[f], idx_v)

            pltpu.async_copy(tabT_hbm.at[r], row_v, sem2).wait()
            for q in range(NQ):
                k = (j * NQ + q) % 2
                if desc[k] is not None:
                    desc[k].wait()
                ov = outs[k]

                @plsc.parallel_loop(0, QTR, step=LANES, unroll=8)
                def _(i, q=q, ov=ov):
                    idxv = idx_v[pl.ds(q * QTR + i, LANES)]
                    ov[pl.ds(i, LANES)] = plsc.load_gather(row_v, [idxv])

                desc[k] = pltpu.async_copy(
                    ov, out_hbm.at[r, pl.ds(q * QTR, QTR)], sems[k]
                )
        for d in desc:
            if d is not None:
                d.wait()

    return _sc_lookup


_BN = 1024


def _mlp_body(xnT, eT, w1nT, w1eT, b1, g1, be1, w2T, b2, g2, be2,
              w3T, b3, g3, be3, w4T, b4, out):
    inv = 1.0 / (1.0 + EPS) ** 0.5
    bf = jnp.bfloat16
    dot = lambda a, b: jnp.dot(
        a.astype(bf), b.astype(bf), preferred_element_type=jnp.float32
    )
    h = dot(w1eT[...], eT[...])
    h = h + jnp.dot(w1nT[...], xnT[...], preferred_element_type=jnp.float32)
    h = jnp.maximum(h + b1[...], 0.0) * (g1[...] * inv) + be1[...]
    h = dot(w2T[...], h)
    h = jnp.maximum(h + b2[...], 0.0) * (g2[...] * inv) + be2[...]
    h = dot(w3T[...], h)
    h = jnp.maximum(h + b3[...], 0.0) * (g3[...] * inv) + be3[...]
    out[...] = jnp.dot(w4T[...], h, preferred_element_type=jnp.float32) + b4[...]


def _mlp(xnT, eT, w1nT, w1eT, b1, g1, be1, w2T, b2, g2, be2,
         w3T, b3, g3, be3, w4T, b4):
    full = lambda r, c: pl.BlockSpec((r, c), lambda i: (0, 0))
    col = lambda r: pl.BlockSpec((r, _BN), lambda i: (0, i))
    return pl.pallas_call(
        _mlp_body,
        grid=(B // _BN,),
        in_specs=[
            col(NUM_NUM), col(ROWS),
            full(1024, NUM_NUM), full(1024, ROWS),
            full(1024, 1), full(1024, 1), full(1024, 1),
            full(512, 1024), full(512, 1), full(512, 1), full(512, 1),
            full(256, 512), full(256, 1), full(256, 1), full(256, 1),
            full(1, 256), full(1, 1),
        ],
        out_specs=col(1),
        out_shape=jax.ShapeDtypeStruct((1, B), jnp.float32),
        compiler_params=pltpu.CompilerParams(
            dimension_semantics=("arbitrary",)
        ),
    )(xnT, eT, w1nT, w1eT, b1, g1, be1, w2T, b2, g2, be2,
      w3T, b3, g3, be3, w4T, b4)


def kernel(x_num, x_cat, tables, W1, b1, g1, be1, W2, b2, g2, be2,
           W3, b3, g3, be3, W4, b4):
    tabT = tables.transpose(0, 2, 1).reshape(ROWS, VOCAB)
    xcatT = x_cat.T
    embT = _make_sc_lookup()(tabT, xcatT)
    c = lambda v: v.reshape(-1, 1)
    out = _mlp(x_num.T, embT,
               W1[:NUM_NUM].T, W1[NUM_NUM:].T, c(b1), c(g1), c(be1),
               W2.T, c(b2), c(g2), c(be2), W3.T, c(b3), c(g3), c(be3),
               W4.T, c(b4))
    return out[0]


# speedup vs baseline: 1.0076x; 1.0076x over previous
import functools

import jax
import jax.numpy as jnp
from jax import lax
from jax.experimental import pallas as pl
from jax.experimental.pallas import tpu as pltpu
from jax.experimental.pallas import tpu_sc as plsc

NUM_FIELDS = 26
VOCAB = 100000
EMB = 32
B = 16384
NUM_NUM = 13
EPS = 1e-5

NC = 2
NS = 16
NW = NC * NS
ROWS = NUM_FIELDS * EMB
RPW = ROWS // NW
NQ = 4
QTR = B // NQ
LANES = 16


@functools.cache
def _make_sc_lookup():
    mesh = plsc.VectorSubcoreMesh(
        core_axis_name="c", subcore_axis_name="s", num_cores=NC, num_subcores=NS
    )

    @functools.partial(
        pl.kernel,
        out_type=jax.ShapeDtypeStruct((ROWS, B), jnp.float32),
        mesh=mesh,
        scratch_types=[
            pltpu.VMEM((VOCAB,), jnp.float32),
            pltpu.VMEM((B,), jnp.int32),
            pltpu.VMEM((QTR,), jnp.float32),
            pltpu.VMEM((QTR,), jnp.float32),
            pltpu.SemaphoreType.DMA,
            pltpu.SemaphoreType.DMA,
            pltpu.SemaphoreType.DMA,
        ],
        compiler_params=pltpu.CompilerParams(
            use_tc_tiling_on_sc=True, needs_layout_passes=False
        ),
    )
    def _sc_lookup(tabT_hbm, xcatT_hbm, out_hbm, row_v, idx_v,
                   out_v0, out_v1, sem0, sem1, sem2):
        wid = lax.axis_index("s") * NC + lax.axis_index("c")
        r0 = wid * RPW
        outs = (out_v0, out_v1)
        sems = (sem0, sem1)
        desc = [None, None]
        for j in range(RPW):
            r = r0 + j
            f = r // EMB
            if j == 0:
                pltpu.sync_copy(xcatT_hbm.at[f], idx_v)
            else:
                fprev = (r - 1) // EMB

                @pl.when(f != fprev)
                def _():
                    pltpu.sync_copy(xcatT_hbm.at[f], idx_v)

            pltpu.async_copy(tabT_hbm.at[r], row_v, sem2).wait()
            for q in range(NQ):
                k = (j * NQ + q) % 2
                if desc[k] is not None:
                    desc[k].wait()
                ov = outs[k]

                @plsc.parallel_loop(0, QTR, step=LANES, unroll=8)
                def _(i, q=q, ov=ov):
                    idxv = idx_v[pl.ds(q * QTR + i, LANES)]
                    ov[pl.ds(i, LANES)] = plsc.load_gather(row_v, [idxv])

                desc[k] = pltpu.async_copy(
                    ov, out_hbm.at[r, pl.ds(q * QTR, QTR)], sems[k]
                )
        for d in desc:
            if d is not None:
                d.wait()

    return _sc_lookup


_BN = 2048


def _mlp_body(xnT, eT, w1nT, w1eT, b1, g1, be1, w2T, b2, g2, be2,
              w3T, b3, g3, be3, w4T, b4, out):
    inv = 1.0 / (1.0 + EPS) ** 0.5
    h = jnp.dot(w1eT[...], eT[...], preferred_element_type=jnp.float32)
    h = h + jnp.dot(w1nT[...], xnT[...], preferred_element_type=jnp.float32)
    h = jnp.maximum(h + b1[...], 0.0) * (g1[...] * inv) + be1[...]
    h = jnp.dot(w2T[...], h, preferred_element_type=jnp.float32)
    h = jnp.maximum(h + b2[...], 0.0) * (g2[...] * inv) + be2[...]
    h = jnp.dot(w3T[...], h, preferred_element_type=jnp.float32)
    h = jnp.maximum(h + b3[...], 0.0) * (g3[...] * inv) + be3[...]
    out[...] = jnp.dot(w4T[...], h, preferred_element_type=jnp.float32) + b4[...]


def _mlp(xnT, eT, w1nT, w1eT, b1, g1, be1, w2T, b2, g2, be2,
         w3T, b3, g3, be3, w4T, b4):
    full = lambda r, c: pl.BlockSpec((r, c), lambda i: (0, 0))
    col = lambda r: pl.BlockSpec((r, _BN), lambda i: (0, i))
    return pl.pallas_call(
        _mlp_body,
        grid=(B // _BN,),
        in_specs=[
            col(NUM_NUM), col(ROWS),
            full(1024, NUM_NUM), full(1024, ROWS),
            full(1024, 1), full(1024, 1), full(1024, 1),
            full(512, 1024), full(512, 1), full(512, 1), full(512, 1),
            full(256, 512), full(256, 1), full(256, 1), full(256, 1),
            full(1, 256), full(1, 1),
        ],
        out_specs=col(1),
        out_shape=jax.ShapeDtypeStruct((1, B), jnp.float32),
        compiler_params=pltpu.CompilerParams(
            dimension_semantics=("arbitrary",)
        ),
    )(xnT, eT, w1nT, w1eT, b1, g1, be1, w2T, b2, g2, be2,
      w3T, b3, g3, be3, w4T, b4)


def kernel(x_num, x_cat, tables, W1, b1, g1, be1, W2, b2, g2, be2,
           W3, b3, g3, be3, W4, b4):
    tabT = tables.transpose(0, 2, 1).reshape(ROWS, VOCAB)
    xcatT = x_cat.T
    embT = _make_sc_lookup()(tabT, xcatT)
    c = lambda v: v.reshape(-1, 1)
    out = _mlp(x_num.T, embT,
               W1[:NUM_NUM].T, W1[NUM_NUM:].T, c(b1), c(g1), c(be1),
               W2.T, c(b2), c(g2), c(be2), W3.T, c(b3), c(g3), c(be3),
               W4.T, c(b4))
    return out[0]
